# baseline (speedup 1.0000x reference)
import functools

import jax
import jax.numpy as jnp
from jax import lax
from jax.experimental import pallas as pl
from jax.experimental.pallas import tpu as pltpu
from jax.experimental.pallas import tpu_sc as plsc

_R = 64 * 768
_HW = 256
_NW = 32
_RPW = _R // _NW
_CH = 64
_NCH = _RPW // _CH
_NG = _HW // 16


def _permute_body(in_hbm, p_hbm, out_hbm, p_v, in_v, out_v):
    wid = lax.axis_index("s") * 2 + lax.axis_index("c")
    base = wid * _RPW * _HW

    pltpu.sync_copy(p_hbm, p_v)
    col = [p_v[pl.ds(16 * j, 16)] for j in range(_NG)]

    def chunk_body(ci, carry):
        off = base + ci * (_CH * _HW)
        pltpu.sync_copy(in_hbm.at[pl.ds(off, _CH * _HW)], in_v)

        def row_body(t, c2):
            toff = t * _HW
            for j in range(_NG):
                g = plsc.load_gather(in_v, [col[j] + toff])
                out_v[pl.ds(toff + 16 * j, 16)] = g
            return c2

        lax.fori_loop(0, _CH, row_body, 0, unroll=2)
        pltpu.sync_copy(out_v, out_hbm.at[pl.ds(off, _CH * _HW)])
        return carry

    lax.fori_loop(0, _NCH, chunk_body, 0)


@functools.partial(jax.jit, static_argnames=())
def _permute(x_flat, p):
    mesh = plsc.VectorSubcoreMesh(core_axis_name="c", subcore_axis_name="s")
    f = pl.kernel(
        _permute_body,
        mesh=mesh,
        compiler_params=pltpu.CompilerParams(needs_layout_passes=False),
        out_type=jax.ShapeDtypeStruct((_R * _HW,), jnp.float32),
        scratch_types=[
            pltpu.VMEM((_HW,), jnp.int32),
            pltpu.VMEM((_CH * _HW,), jnp.float32),
            pltpu.VMEM((_CH * _HW,), jnp.float32),
        ],
    )
    return f(x_flat, p)


def kernel(inputs, p_array):
    B, C, H, W = inputs.shape
    x = inputs.reshape(B * C * H * W)
    p = p_array.astype(jnp.int32)
    out = _permute(x, p)
    return out.reshape(B, C, H, W)


# speedup vs baseline: 1.0409x; 1.0409x over previous
import jax
import jax.numpy as jnp
from jax import lax
from jax.experimental import pallas as pl
from jax.experimental.pallas import tpu as pltpu
from jax.experimental.pallas import tpu_sc as plsc

_R = 64 * 768
_HW = 256
_NW = 32
_RPW = _R // _NW
_CH = 96
_NCH = _RPW // _CH
_NG = _HW // 16


def _permute_body(in_hbm, p_hbm, out_hbm, p_v,
                  in_v0, in_v1, out_v0, out_v1, si0, si1, so0, so1):
    wid = lax.axis_index("s") * 2 + lax.axis_index("c")
    base = wid * _RPW * _HW

    pltpu.sync_copy(p_hbm, p_v)
    col = [p_v[pl.ds(16 * j, 16)] for j in range(_NG)]

    in_bufs, out_bufs = (in_v0, in_v1), (out_v0, out_v1)
    sin, sout = (si0, si1), (so0, so1)

    def start_in(ci, k):
        return pltpu.async_copy(
            in_hbm.at[pl.ds(base + ci * _CH * _HW, _CH * _HW)],
            in_bufs[k], sin[k])

    def start_out(ci, k):
        return pltpu.async_copy(
            out_bufs[k],
            out_hbm.at[pl.ds(base + ci * _CH * _HW, _CH * _HW)], sout[k])

    in_d = {0: start_in(0, 0), 1: start_in(1, 1)}
    out_d = {}
    for ci in range(_NCH):
        k = ci & 1
        in_d[ci].wait()
        if ci >= 2:
            out_d[ci - 2].wait()
        iv, ov = in_bufs[k], out_bufs[k]

        def row_body(t, c, iv=iv, ov=ov):
            toff = t * _HW
            for j in range(_NG):
                g = plsc.load_gather(iv, [col[j] + toff])
                ov[pl.ds(toff + 16 * j, 16)] = g
            return c

        lax.fori_loop(0, _CH, row_body, 0, unroll=4)
        out_d[ci] = start_out(ci, k)
        if ci + 2 < _NCH:
            in_d[ci + 2] = start_in(ci + 2, k)
    out_d[_NCH - 2].wait()
    out_d[_NCH - 1].wait()


@jax.jit
def _permute(x2, p):
    mesh = plsc.VectorSubcoreMesh(core_axis_name="c", subcore_axis_name="s")
    f = pl.kernel(
        _permute_body,
        mesh=mesh,
        compiler_params=pltpu.CompilerParams(needs_layout_passes=False),
        out_type=jax.ShapeDtypeStruct((_R * _HW,), jnp.float32),
        scratch_types=[
            pltpu.VMEM((_HW,), jnp.int32),
            pltpu.VMEM((_CH * _HW,), jnp.float32),
            pltpu.VMEM((_CH * _HW,), jnp.float32),
            pltpu.VMEM((_CH * _HW,), jnp.float32),
            pltpu.VMEM((_CH * _HW,), jnp.float32),
            pltpu.SemaphoreType.DMA,
            pltpu.SemaphoreType.DMA,
            pltpu.SemaphoreType.DMA,
            pltpu.SemaphoreType.DMA,
        ],
    )
    return f(x2, p)


def kernel(inputs, p_array):
    B, C, H, W = inputs.shape
    x = inputs.reshape(B * C * H * W)
    p = p_array.astype(jnp.int32)
    out = _permute(x, p)
    return out.reshape(B, C, H, W)


# speedup vs baseline: 2.0220x; 1.9425x over previous
import jax
import jax.numpy as jnp
from jax import lax
from jax.experimental import pallas as pl
from jax.experimental.pallas import tpu as pltpu
from jax.experimental.pallas import tpu_sc as plsc

_B = 64
_C = 768
_HW = 256
_CT = _C // 128
_NW = 32
_PPW = _B * _CT // _NW


def _permute_body(in_hbm, p_hbm, out_hbm, p_v,
                  in_v0, in_v1, ov0, ov1, si0, si1, so0, so1):
    wid = lax.axis_index("s") * 2 + lax.axis_index("c")

    pltpu.sync_copy(p_hbm, p_v)

    in_bufs, sin = (in_v0, in_v1), (si0, si1)
    out_bufs, sout = (ov0, ov1), (so0, so1)

    def start_in(i, k):
        b = _PPW // _CT * wid + i // _CT
        c0 = (i % _CT) * 128
        return pltpu.async_copy(
            in_hbm.at[b, :, pl.ds(c0, 128)], in_bufs[k], sin[k])

    def start_out(i, h):
        b = _PPW // _CT * wid + i // _CT
        c0 = (i % _CT) * 128
        return pltpu.async_copy(
            out_bufs[h],
            out_hbm.at[b, pl.ds(h * 128, 128), pl.ds(c0, 128)], sout[h])

    zv = jnp.zeros((16,), jnp.int32)
    iota = lax.iota(jnp.int32, 16)

    in_d = {0: start_in(0, 0), 1: start_in(1, 1)}
    out_d = {}
    for i in range(_PPW):
        k = i & 1
        in_d[i].wait()
        iv = in_bufs[k]
        for h in range(2):
            if i >= 1:
                out_d[(i - 1, h)].wait()
            ov = out_bufs[h]

            def g_body(g, cc, iv=iv, ov=ov, h=h):
                pv = p_v[pl.ds(h * 128 + g * 16, 16)]
                sv = iota + g * 16

                def col_body(c, c2):
                    cs = c + zv
                    vals = plsc.load_gather(iv, [pv, cs])
                    plsc.store_scatter(ov, [sv, cs], vals)
                    return c2

                lax.fori_loop(0, 128, col_body, 0, unroll=8)
                return cc

            lax.fori_loop(0, 8, g_body, 0)
            out_d[(i, h)] = start_out(i, h)
        if i + 2 < _PPW:
            in_d[i + 2] = start_in(i + 2, k)
    out_d[(_PPW - 1, 0)].wait()
    out_d[(_PPW - 1, 1)].wait()


@jax.jit
def _permute(x3, p):
    mesh = plsc.VectorSubcoreMesh(core_axis_name="c", subcore_axis_name="s")
    f = pl.kernel(
        _permute_body,
        mesh=mesh,
        compiler_params=pltpu.CompilerParams(needs_layout_passes=False),
        out_type=jax.ShapeDtypeStruct((_B, _HW, _C), jnp.float32),
        scratch_types=[
            pltpu.VMEM((_HW,), jnp.int32),
            pltpu.VMEM((_HW, 128), jnp.float32),
            pltpu.VMEM((_HW, 128), jnp.float32),
            pltpu.VMEM((128, 128), jnp.float32),
            pltpu.VMEM((128, 128), jnp.float32),
            pltpu.SemaphoreType.DMA,
            pltpu.SemaphoreType.DMA,
            pltpu.SemaphoreType.DMA,
            pltpu.SemaphoreType.DMA,
        ],
    )
    return f(x3, p)


def kernel(inputs, p_array):
    B, C, H, W = inputs.shape
    x3 = jnp.transpose(inputs, (0, 2, 3, 1)).reshape(B, H * W, C)
    p = p_array.astype(jnp.int32)
    out3 = _permute(x3, p)
    return jnp.transpose(out3.reshape(B, H, W, C), (0, 3, 1, 2))


# speedup vs baseline: 17.3271x; 8.5695x over previous
import jax
import jax.numpy as jnp
from jax import lax
from jax.experimental import pallas as pl
from jax.experimental.pallas import tpu as pltpu
from jax.experimental.pallas import tpu_sc as plsc

_B = 64
_C = 768
_HW = 256
_NW = 32
_CH = 64
_NCHB = _HW // _CH
_CPW = _B * _NCHB // _NW


def _permute_body(in_hbm, p_hbm, out_hbm, p_v, r0, r1, sg0, sg1, so0, so1):
    wid = lax.axis_index("s") * 2 + lax.axis_index("c")

    pltpu.sync_copy(p_hbm, p_v)

    bufs, sg, so = (r0, r1), (sg0, sg1), (so0, so1)

    def start_gather(i, k):
        b = wid * (_CPW // _NCHB) + i // _NCHB
        s0 = (i % _NCHB) * _CH
        return pltpu.async_copy(
            in_hbm.at[b].at[p_v.at[pl.ds(s0, _CH)]], bufs[k], sg[k])

    def start_out(i, k):
        b = wid * (_CPW // _NCHB) + i // _NCHB
        s0 = (i % _NCHB) * _CH
        return pltpu.async_copy(
            bufs[k], out_hbm.at[b, pl.ds(s0, _CH)], so[k])

    gd = {0: start_gather(0, 0), 1: start_gather(1, 1)}
    od = {}
    for i in range(_CPW):
        k = i & 1
        gd[i].wait()
        od[i] = start_out(i, k)
        if i + 2 < _CPW:
            od[i].wait()
            gd[i + 2] = start_gather(i + 2, k)
    od[_CPW - 2].wait()
    od[_CPW - 1].wait()


@jax.jit
def _permute(x3, p):
    mesh = plsc.VectorSubcoreMesh(core_axis_name="c", subcore_axis_name="s")
    f = pl.kernel(
        _permute_body,
        mesh=mesh,
        compiler_params=pltpu.CompilerParams(needs_layout_passes=False),
        out_type=jax.ShapeDtypeStruct((_B, _HW, _C), jnp.float32),
        scratch_types=[
            pltpu.VMEM((_HW,), jnp.int32),
            pltpu.VMEM((_CH, _C), jnp.float32),
            pltpu.VMEM((_CH, _C), jnp.float32),
            pltpu.SemaphoreType.DMA,
            pltpu.SemaphoreType.DMA,
            pltpu.SemaphoreType.DMA,
            pltpu.SemaphoreType.DMA,
        ],
    )
    return f(x3, p)


def kernel(inputs, p_array):
    B, C, H, W = inputs.shape
    x3 = jnp.transpose(inputs, (0, 2, 3, 1)).reshape(B, H * W, C)
    p = p_array.astype(jnp.int32)
    out3 = _permute(x3, p)
    return jnp.transpose(out3.reshape(B, H, W, C), (0, 3, 1, 2))


# speedup vs baseline: 17.5229x; 1.0113x over previous
import jax
import jax.numpy as jnp
from jax import lax
from jax.experimental import pallas as pl
from jax.experimental.pallas import tpu as pltpu
from jax.experimental.pallas import tpu_sc as plsc

_B = 64
_C = 768
_HW = 256
_NW = 32
_CH = 16
_NCHB = _HW // _CH
_CPW = _B * _NCHB // _NW
_NBUF = 8
_LEAD = 4


def _permute_body(in_hbm, p_hbm, out_hbm, p_v, *rest):
    bufs, sg, so = rest[:_NBUF], rest[_NBUF:2 * _NBUF], rest[2 * _NBUF:]
    wid = lax.axis_index("s") * 2 + lax.axis_index("c")

    pltpu.sync_copy(p_hbm, p_v)

    def start_gather(i):
        k = i % _NBUF
        b = wid * (_CPW // _NCHB) + i // _NCHB
        s0 = (i % _NCHB) * _CH
        return pltpu.async_copy(
            in_hbm.at[b].at[p_v.at[pl.ds(s0, _CH)]], bufs[k], sg[k])

    def start_out(i):
        k = i % _NBUF
        b = wid * (_CPW // _NCHB) + i // _NCHB
        s0 = (i % _NCHB) * _CH
        return pltpu.async_copy(
            bufs[k], out_hbm.at[b, pl.ds(s0, _CH)], so[k])

    gd = {i: start_gather(i) for i in range(_LEAD)}
    od = {}
    for i in range(_CPW):
        gd[i].wait()
        od[i] = start_out(i)
        j = i + _LEAD
        if j < _CPW:
            if j >= _NBUF:
                od[j - _NBUF].wait()
            gd[j] = start_gather(j)
    for i in range(_CPW - _NBUF, _CPW):
        od[i].wait()


@jax.jit
def _permute(x3, p):
    mesh = plsc.VectorSubcoreMesh(core_axis_name="c", subcore_axis_name="s")
    f = pl.kernel(
        _permute_body,
        mesh=mesh,
        compiler_params=pltpu.CompilerParams(needs_layout_passes=False),
        out_type=jax.ShapeDtypeStruct((_B, _HW, _C), jnp.float32),
        scratch_types=(
            [pltpu.VMEM((_HW,), jnp.int32)]
            + [pltpu.VMEM((_CH, _C), jnp.float32)] * _NBUF
            + [pltpu.SemaphoreType.DMA] * (2 * _NBUF)
        ),
    )
    return f(x3, p)


def kernel(inputs, p_array):
    B, C, H, W = inputs.shape
    x3 = jnp.transpose(inputs, (0, 2, 3, 1)).reshape(B, H * W, C)
    p = p_array.astype(jnp.int32)
    out3 = _permute(x3, p)
    return jnp.transpose(out3.reshape(B, H, W, C), (0, 3, 1, 2))
